# baseline (speedup 1.0000x reference)
import functools

import jax
import jax.numpy as jnp
from jax import lax
from jax.experimental import pallas as pl
from jax.experimental.pallas import tpu as pltpu
from jax.experimental.pallas import tpu_sc as plsc

D = 64
L = 50
PAD = 999999
CT = 4
TOK = CT * L
GSUB = 100
NG = TOK // GSUB
UNROLL = 5


def _permute(x, idx):
    return lax.gather(
        x, idx[:, None],
        lax.GatherDimensionNumbers(
            offset_dims=(), collapsed_slice_dims=(0,), start_index_map=(0,)),
        slice_sizes=(1,),
        mode=lax.GatherScatterMode.PROMISE_IN_BOUNDS)


def _hsum_all(x, lanes):
    for s in (8, 4, 2, 1):
        x = x + _permute(x, lanes ^ s)
    return x


def _rsqrt(x, iters=2):
    half = x * 0.5
    i = lax.bitcast_convert_type(x, jnp.int32)
    i = 0x5F3759DF - lax.shift_right_logical(i, 1)
    y = lax.bitcast_convert_type(i, jnp.float32)
    for _ in range(iters):
        y = y * (1.5 - half * y * y)
    return y


def kernel(input_ids, attention_mask, W):
    B, N, Lx = input_ids.shape
    T = B * N
    ids_flat = input_ids.reshape(T * L)
    mask_flat = attention_mask.reshape(T * L)

    info = plsc.get_sparse_core_info()
    NC, NS = info.num_cores, info.num_subcores
    NW = NC * NS
    tw_per_w = T // NW
    n_chunks = tw_per_w // CT
    tok_per_w = tw_per_w * L

    mesh = plsc.VectorSubcoreMesh(core_axis_name="c", subcore_axis_name="s")

    @functools.partial(
        pl.kernel,
        mesh=mesh,
        out_type=jax.ShapeDtypeStruct((T, D), jnp.float32),
        scratch_types=[
            pltpu.VMEM((tok_per_w,), jnp.int32),
            pltpu.VMEM((tok_per_w,), jnp.float32),
            pltpu.VMEM((TOK, D), jnp.float32),
            pltpu.VMEM((TOK, D), jnp.float32),
            pltpu.VMEM((TOK, D), jnp.float32),
            pltpu.VMEM((TOK, D), jnp.float32),
            pltpu.VMEM((CT, D), jnp.float32),
            pltpu.SemaphoreType.DMA,
            pltpu.SemaphoreType.DMA,
            pltpu.SemaphoreType.DMA,
            pltpu.SemaphoreType.DMA,
        ],
        compiler_params=pltpu.CompilerParams(
            needs_layout_passes=False, use_tc_tiling_on_sc=False),
    )
    def sc_kernel(ids_hbm, mask_hbm, w_hbm, out_hbm,
                  ids_v, mask_v, rows0, rows1, rows2, rows3, out_v,
                  sem0, sem1, sem2, sem3):
        wid = lax.axis_index("s") * NC + lax.axis_index("c")
        tw0 = wid * tw_per_w
        pltpu.sync_copy(
            ids_hbm.at[pl.ds(pl.multiple_of(wid * tok_per_w, 8), tok_per_w)],
            ids_v)
        pltpu.sync_copy(
            mask_hbm.at[pl.ds(pl.multiple_of(wid * tok_per_w, 8), tok_per_w)],
            mask_v)

        lanes = lax.iota(jnp.int32, 16)
        bufs = (rows0, rows1, rows2, rows3)
        sems = (sem0, sem1, sem2, sem3)
        NB = len(bufs)

        def fire(ci, buf, sem):
            pltpu.async_copy(
                w_hbm.at[ids_v.at[pl.ds(ci * TOK, TOK)]], buf, sem)

        def drain(ci, buf, sem):
            pltpu.make_async_copy(
                w_hbm.at[ids_v.at[pl.ds(ci * TOK, TOK)]], buf, sem).wait()

        for b in range(NB - 1):
            fire(b, bufs[b], sems[b])

        def do_chunk(ci, par):
            buf, sem = bufs[par], sems[par]
            nxt_par = (par + NB - 1) % NB
            nxt = jnp.minimum(ci + NB - 1, n_chunks - 1)
            fire(nxt, bufs[nxt_par], sems[nxt_par])
            drain(ci, buf, sem)

            def tweet_body(t, _):
                def tok_group(j, carry):
                    a0, a1, a2, a3 = carry
                    for k in range(UNROLL):
                        row = t * L + j * UNROLL + k
                        v0 = buf[row, pl.ds(0, 16)]
                        v1 = buf[row, pl.ds(16, 16)]
                        v2 = buf[row, pl.ds(32, 16)]
                        v3 = buf[row, pl.ds(48, 16)]
                        p = v0 * v0 + v1 * v1 + v2 * v2 + v3 * v3
                        mv = plsc.load_gather(
                            mask_v,
                            [lax.broadcast(ci * TOK + row, (16,))])
                        r = _rsqrt(_hsum_all(p, lanes), iters=1) * mv
                        a0 = a0 + v0 * r
                        a1 = a1 + v1 * r
                        a2 = a2 + v2 * r
                        a3 = a3 + v3 * r
                    return (a0, a1, a2, a3)

                z = jnp.zeros((16,), jnp.float32)
                a0, a1, a2, a3 = lax.fori_loop(
                    0, L // UNROLL, tok_group, (z, z, z, z))
                mb = ci * TOK + t * L
                m = (mask_v[pl.ds(mb, 16)] + mask_v[pl.ds(mb + 16, 16)]
                     + mask_v[pl.ds(mb + 32, 16)]
                     + jnp.where(lanes >= 14, mask_v[pl.ds(mb + 34, 16)], 0.0))
                sl = _hsum_all(m, lanes)
                inv = jnp.where(sl > 0.0, 1.0 / sl, 0.0)
                out_v[t, pl.ds(0, 16)] = a0 * inv
                out_v[t, pl.ds(16, 16)] = a1 * inv
                out_v[t, pl.ds(32, 16)] = a2 * inv
                out_v[t, pl.ds(48, 16)] = a3 * inv
                return 0

            lax.fori_loop(0, CT, tweet_body, 0)
            pltpu.sync_copy(
                out_v,
                out_hbm.at[pl.ds(pl.multiple_of(tw0 + ci * CT, CT), CT)])

        def quad_body(cq, _):
            for par in range(NB):
                do_chunk(cq * NB + par, par)
            return 0

        lax.fori_loop(0, n_chunks // NB, quad_body, 0)
        for j in range(NB - 1):
            par = (n_chunks + j) % NB
            drain(n_chunks - 1, bufs[par], sems[par])

    out = sc_kernel(ids_flat, mask_flat, W)
    return out.reshape(B, N, D)


# speedup vs baseline: 1.0403x; 1.0403x over previous
import functools

import jax
import jax.numpy as jnp
from jax import lax
from jax.experimental import pallas as pl
from jax.experimental.pallas import tpu as pltpu
from jax.experimental.pallas import tpu_sc as plsc

D = 64
L = 50
PAD = 999999
CT = 8
TOK = CT * L
GSUB = 100
NG = TOK // GSUB
UNROLL = 10


def _permute(x, idx):
    return lax.gather(
        x, idx[:, None],
        lax.GatherDimensionNumbers(
            offset_dims=(), collapsed_slice_dims=(0,), start_index_map=(0,)),
        slice_sizes=(1,),
        mode=lax.GatherScatterMode.PROMISE_IN_BOUNDS)


def _hsum_all(x, lanes):
    for s in (8, 4, 2, 1):
        x = x + _permute(x, lanes ^ s)
    return x


def _rsqrt(x, iters=2):
    half = x * 0.5
    i = lax.bitcast_convert_type(x, jnp.int32)
    i = 0x5F3759DF - lax.shift_right_logical(i, 1)
    y = lax.bitcast_convert_type(i, jnp.float32)
    for _ in range(iters):
        y = y * (1.5 - half * y * y)
    return y


def kernel(input_ids, attention_mask, W):
    B, N, Lx = input_ids.shape
    T = B * N
    ids_flat = input_ids.reshape(T * L)
    mask_flat = attention_mask.reshape(T * L)

    info = plsc.get_sparse_core_info()
    NC, NS = info.num_cores, info.num_subcores
    NW = NC * NS
    tw_per_w = T // NW
    n_chunks = tw_per_w // CT
    tok_per_w = tw_per_w * L

    mesh = plsc.VectorSubcoreMesh(core_axis_name="c", subcore_axis_name="s")

    @functools.partial(
        pl.kernel,
        mesh=mesh,
        out_type=jax.ShapeDtypeStruct((T, D), jnp.float32),
        scratch_types=[
            pltpu.VMEM((tok_per_w,), jnp.int32),
            pltpu.VMEM((tok_per_w,), jnp.float32),
            pltpu.VMEM((TOK, D), jnp.float32),
            pltpu.VMEM((TOK, D), jnp.float32),
            pltpu.VMEM((CT, D), jnp.float32),
            pltpu.SemaphoreType.DMA,
            pltpu.SemaphoreType.DMA,
        ],
        compiler_params=pltpu.CompilerParams(
            needs_layout_passes=False, use_tc_tiling_on_sc=False),
    )
    def sc_kernel(ids_hbm, mask_hbm, w_hbm, out_hbm,
                  ids_v, mask_v, rows0, rows1, out_v, sem0, sem1):
        wid = lax.axis_index("s") * NC + lax.axis_index("c")
        tw0 = wid * tw_per_w
        pltpu.sync_copy(
            ids_hbm.at[pl.ds(pl.multiple_of(wid * tok_per_w, 8), tok_per_w)],
            ids_v)
        pltpu.sync_copy(
            mask_hbm.at[pl.ds(pl.multiple_of(wid * tok_per_w, 8), tok_per_w)],
            mask_v)

        lanes = lax.iota(jnp.int32, 16)
        bufs = (rows0, rows1)
        sems = (sem0, sem1)
        NB = len(bufs)

        def fire(ci, buf, sem):
            pltpu.async_copy(
                w_hbm.at[ids_v.at[pl.ds(ci * TOK, TOK)]], buf, sem)

        def drain(ci, buf, sem):
            pltpu.make_async_copy(
                w_hbm.at[ids_v.at[pl.ds(ci * TOK, TOK)]], buf, sem).wait()

        for b in range(NB - 1):
            fire(b, bufs[b], sems[b])

        def do_chunk(ci, par):
            buf, sem = bufs[par], sems[par]
            nxt_par = (par + NB - 1) % NB
            nxt = jnp.minimum(ci + NB - 1, n_chunks - 1)
            fire(nxt, bufs[nxt_par], sems[nxt_par])
            drain(ci, buf, sem)

            def tweet_body(t, _):
                def tok_group(j, carry):
                    a0, a1, a2, a3 = carry
                    for k in range(UNROLL):
                        row = t * L + j * UNROLL + k
                        v0 = buf[row, pl.ds(0, 16)]
                        v1 = buf[row, pl.ds(16, 16)]
                        v2 = buf[row, pl.ds(32, 16)]
                        v3 = buf[row, pl.ds(48, 16)]
                        p = v0 * v0 + v1 * v1 + v2 * v2 + v3 * v3
                        mv = plsc.load_gather(
                            mask_v,
                            [lax.broadcast(ci * TOK + row, (16,))])
                        r = _rsqrt(_hsum_all(p, lanes), iters=1) * mv
                        a0 = a0 + v0 * r
                        a1 = a1 + v1 * r
                        a2 = a2 + v2 * r
                        a3 = a3 + v3 * r
                    return (a0, a1, a2, a3)

                z = jnp.zeros((16,), jnp.float32)
                a0, a1, a2, a3 = lax.fori_loop(
                    0, L // UNROLL, tok_group, (z, z, z, z))
                mb = ci * TOK + t * L
                m = (mask_v[pl.ds(mb, 16)] + mask_v[pl.ds(mb + 16, 16)]
                     + mask_v[pl.ds(mb + 32, 16)]
                     + jnp.where(lanes >= 14, mask_v[pl.ds(mb + 34, 16)], 0.0))
                sl = _hsum_all(m, lanes)
                inv = jnp.where(sl > 0.0, 1.0 / sl, 0.0)
                out_v[t, pl.ds(0, 16)] = a0 * inv
                out_v[t, pl.ds(16, 16)] = a1 * inv
                out_v[t, pl.ds(32, 16)] = a2 * inv
                out_v[t, pl.ds(48, 16)] = a3 * inv
                return 0

            lax.fori_loop(0, CT, tweet_body, 0)
            pltpu.sync_copy(
                out_v,
                out_hbm.at[pl.ds(pl.multiple_of(tw0 + ci * CT, CT), CT)])

        def quad_body(cq, _):
            for par in range(NB):
                do_chunk(cq * NB + par, par)
            return 0

        lax.fori_loop(0, n_chunks // NB, quad_body, 0)
        for j in range(NB - 1):
            par = (n_chunks + j) % NB
            drain(n_chunks - 1, bufs[par], sems[par])

    out = sc_kernel(ids_flat, mask_flat, W)
    return out.reshape(B, N, D)
